# baseline (speedup 1.0000x reference)
import functools

import numpy as np
import jax
import jax.numpy as jnp
from jax import lax
from jax.experimental import pallas as pl
from jax.experimental.pallas import tpu as pltpu
from jax.experimental.pallas import tpu_sc as plsc

B = 4
L = 2048
H = 16
D = 128
K = 1024
V = 196608
E = 8
EPS = 1e-5
T = B * L


def _primes_from(lo, count):
    def isp(n):
        if n % 2 == 0:
            return n == 2
        i = 3
        while i * i <= n:
            if n % i == 0:
                return False
            i += 2
        return True

    out, x = [], lo
    while len(out) < count:
        if isp(x):
            out.append(x)
        x += 1
    return out


_PRIMES = np.array(_primes_from(V + 2, H), dtype=np.int32)

TLA = 512


CK = 128


def _pq_ids_body(x_ref, mt_ref, ids_ref):
    it = lax.broadcasted_iota(jnp.int32, (TLA, CK), 1)
    for h in range(H):
        xh = x_ref[:, h * D:(h + 1) * D]
        mt = mt_ref[h]
        prod = lax.dot_general(xh, mt, (((1,), (0,)), ((), ())),
                               preferred_element_type=jnp.float32)
        xn2 = jnp.sum(xh * xh, axis=1, keepdims=True)
        mn2 = jnp.sum(mt * mt, axis=0, keepdims=True) * 0.25
        runval = None
        for ci in range(K // CK):
            d = prod[:, ci * CK:(ci + 1) * CK] + xn2
            d = d + mn2[:, ci * CK:(ci + 1) * CK]
            if ci == 0:
                runval, runidx = d, it
            else:
                better = d < runval
                runval = jnp.minimum(runval, d)
                runidx = jnp.where(better, it + ci * CK, runidx)
        m = jnp.min(runval, axis=1, keepdims=True)
        idx = jnp.min(jnp.where(runval == m, runidx, K), axis=1, keepdims=True)
        ids_ref[:, h:h + 1] = idx


NS = 4
TH = T // NS
BPA = TH // TLA


def _make_pq_ids(s):
    return pl.pallas_call(
        _pq_ids_body,
        grid=(BPA,),
        in_specs=[
            pl.BlockSpec((TLA, H * D), lambda i: (s * BPA + i, 0)),
            pl.BlockSpec((H, D, K), lambda i: (0, 0, 0)),
        ],
        out_specs=pl.BlockSpec((TLA, H), lambda i: (i, 0)),
        out_shape=jax.ShapeDtypeStruct((TH, H), jnp.int32),
    )

NW = 32
POS_W = TH // NW
CH = POS_W * H

def _sc_body(ids_hbm, table_hbm, primes_hbm, y_hbm,
             idsbuf, idxbuf, rows, pbuf, sem):
    cid = lax.axis_index("c")
    sid = lax.axis_index("s")
    wid = sid * 2 + cid
    pos0 = wid * POS_W
    base = pos0 * H
    pltpu.sync_copy(ids_hbm.at[pl.ds(base, CH)], idsbuf.at[pl.ds(H, CH)])
    pltpu.sync_copy(primes_hbm, pbuf)
    row_start = (pos0 % L) == 0

    @pl.when(row_start)
    def _():
        idsbuf[pl.ds(0, H)] = jnp.zeros((H,), jnp.int32)

    @pl.when(jnp.logical_not(row_start))
    def _():
        pltpu.sync_copy(ids_hbm.at[pl.ds(base - H, H)], idsbuf.at[pl.ds(0, H)])

    lanes = lax.iota(jnp.int32, H)
    mult = lanes + 1
    primes = pbuf[...]
    offs = lanes * V

    def _hash_step(p, carry):
        cur = idsbuf[pl.ds(p * H + H, H)]
        prev = idsbuf[pl.ds(p * H, H)]
        bg = cur + prev * K
        r = ((bg * mult + mult) % primes) % V + offs
        w0 = (r >> 7) * 1024 + (r & 127)
        pos = lanes * E + p * (H * E)
        for c in range(E):
            plsc.store_scatter(idxbuf, [pos + c], w0 + c * 128)
        return carry

    lax.fori_loop(0, POS_W, _hash_step, 0)
    pltpu.async_copy(table_hbm.at[idxbuf], rows, sem).wait()
    pltpu.sync_copy(rows, y_hbm.at[pl.ds(base * E, CH * E)])


@functools.lru_cache(maxsize=1)
def _make_sc_hash_gather():
    mesh = plsc.VectorSubcoreMesh(core_axis_name="c", subcore_axis_name="s")
    return pl.kernel(
        _sc_body,
        out_type=jax.ShapeDtypeStruct((TH * H * E,), jnp.float32),
        mesh=mesh,
        scratch_types=[
            pltpu.VMEM((CH + H,), jnp.int32),
            pltpu.VMEM((CH * E,), jnp.int32),
            pltpu.VMEM((CH * E,), jnp.float32),
            pltpu.VMEM((H,), jnp.int32),
            pltpu.SemaphoreType.DMA,
        ],
        compiler_params=pltpu.CompilerParams(use_tc_tiling_on_sc=False,
                                             needs_layout_passes=False),
    )


TLC = 512


def _assemble_body(x_ref, y_ref, xs_ref, xb_ref, ys_ref, yb_ref, o_ref):
    y2 = y_ref[...]
    r = lax.broadcasted_iota(jnp.int32, (H * E, H * E), 0)
    c = lax.broadcasted_iota(jnp.int32, (H * E, H * E), 1)
    grp = jnp.where(r // E == c // E, 1.0 / E, 0.0)
    mu = lax.dot_general(y2, grp, (((1,), (0,)), ((), ())),
                         precision=lax.Precision.HIGHEST,
                         preferred_element_type=jnp.float32)
    ms = lax.dot_general(y2 * y2, grp, (((1,), (0,)), ((), ())),
                         precision=lax.Precision.HIGHEST,
                         preferred_element_type=jnp.float32)
    var = ms - mu * mu
    yn = (y2 - mu) * lax.rsqrt(var + EPS) * ys_ref[0, :] + yb_ref[0, :]
    lane = lax.broadcasted_iota(jnp.int32, (TLC, D), 1)
    for h in range(H):
        xh = x_ref[:, h * D:(h + 1) * D]
        mux = jnp.mean(xh, axis=1, keepdims=True)
        varx = jnp.mean(xh * xh, axis=1, keepdims=True) - mux * mux
        xn = ((xh - mux) * lax.rsqrt(varx + EPS)
              * xs_ref[0, h * D:(h + 1) * D] + xb_ref[0, h * D:(h + 1) * D])
        yh = pltpu.roll(yn, (D - E) - E * h, axis=1)
        o_ref[:, h * D:(h + 1) * D] = jnp.where(lane < D - E, xn, yh)


def _assemble_body_prev(x_ref, y_ref, xs_ref, xb_ref, ys_ref, yb_ref,
                        prev_ref, o_ref):
    del prev_ref
    _assemble_body(x_ref, y_ref, xs_ref, xb_ref, ys_ref, yb_ref, o_ref)


BPC = TH // TLC


def _make_assemble(s):
    in_specs = [
        pl.BlockSpec((TLC, H * D), lambda i: (s * BPC + i, 0)),
        pl.BlockSpec((TLC, H * E), lambda i: (i, 0)),
        pl.BlockSpec((1, H * D), lambda i: (0, 0)),
        pl.BlockSpec((1, H * D), lambda i: (0, 0)),
        pl.BlockSpec((1, H * E), lambda i: (0, 0)),
        pl.BlockSpec((1, H * E), lambda i: (0, 0)),
    ]
    body = _assemble_body
    aliases = {}
    if s > 0:
        in_specs.append(pl.BlockSpec((8, H * D), lambda i: (0, 0)))
        body = _assemble_body_prev
        aliases = {6: 0}
    return pl.pallas_call(
        body,
        grid=(BPC,),
        in_specs=in_specs,
        out_specs=pl.BlockSpec((TLC, H * D), lambda i: (s * BPC + i, 0)),
        out_shape=jax.ShapeDtypeStruct((T, H * D), jnp.float32),
        input_output_aliases=aliases,
    )


def kernel(x, means, ngram_table, ln_x_scale, ln_x_bias, ln_y_scale, ln_y_bias):
    x2 = x.reshape(T, H * D)
    meansT = jnp.swapaxes(means, 1, 2) * (-2.0)
    primes = jnp.asarray(_PRIMES)
    table_flat = ngram_table.reshape(V * H // 128, 128, E).transpose(0, 2, 1).reshape(-1)
    xs_ = ln_x_scale.reshape(1, H * D)
    xb_ = ln_x_bias.reshape(1, H * D)
    ys_ = ln_y_scale.reshape(1, H * E)
    yb_ = ln_y_bias.reshape(1, H * E)
    sc = _make_sc_hash_gather()
    ys = []
    for s in range(NS):
        ids = _make_pq_ids(s)(x2, meansT)
        ys.append(sc(ids.reshape(TH * H), table_flat, primes))
    out = None
    for s in range(NS):
        y2 = ys[s].reshape(TH, H * E)
        args = (x2, y2, xs_, xb_, ys_, yb_)
        if s > 0:
            args = args + (out,)
        out = _make_assemble(s)(*args)
    return out.reshape(B, L, H * D)


# speedup vs baseline: 1.1591x; 1.1591x over previous
import functools

import numpy as np
import jax
import jax.numpy as jnp
from jax import lax
from jax.experimental import pallas as pl
from jax.experimental.pallas import tpu as pltpu
from jax.experimental.pallas import tpu_sc as plsc

B = 4
L = 2048
H = 16
D = 128
K = 1024
V = 196608
E = 8
EPS = 1e-5
T = B * L


def _primes_from(lo, count):
    def isp(n):
        if n % 2 == 0:
            return n == 2
        i = 3
        while i * i <= n:
            if n % i == 0:
                return False
            i += 2
        return True

    out, x = [], lo
    while len(out) < count:
        if isp(x):
            out.append(x)
        x += 1
    return out


_PRIMES = np.array(_primes_from(V + 2, H), dtype=np.int32)

TLA = 512


CK = 128


def _pq_ids_body(x_ref, mt_ref, ids_ref):
    it = lax.broadcasted_iota(jnp.int32, (TLA, CK), 1)
    for h in range(H):
        xh = x_ref[:, h * D:(h + 1) * D]
        mt = mt_ref[h]
        prod = lax.dot_general(xh, mt, (((1,), (0,)), ((), ())),
                               preferred_element_type=jnp.float32)
        mn2 = jnp.sum(mt * mt, axis=0, keepdims=True) * 0.25
        runval = None
        for ci in range(K // CK):
            d = prod[:, ci * CK:(ci + 1) * CK] + mn2[:, ci * CK:(ci + 1) * CK]
            if ci == 0:
                runval, runidx = d, it
            else:
                better = d < runval
                runval = jnp.minimum(runval, d)
                runidx = jnp.where(better, it + ci * CK, runidx)
        m = jnp.min(runval, axis=1, keepdims=True)
        idx = jnp.min(jnp.where(runval == m, runidx, K), axis=1, keepdims=True)
        ids_ref[:, h:h + 1] = idx


NS = 2
TH = T // NS
BPA = TH // TLA


def _make_pq_ids(s):
    return pl.pallas_call(
        _pq_ids_body,
        grid=(BPA,),
        in_specs=[
            pl.BlockSpec((TLA, H * D), lambda i: (s * BPA + i, 0)),
            pl.BlockSpec((H, D, K), lambda i: (0, 0, 0)),
        ],
        out_specs=pl.BlockSpec((TLA, H), lambda i: (i, 0)),
        out_shape=jax.ShapeDtypeStruct((TH, H), jnp.int32),
    )

NW = 32
POS_W = TH // NW
CH = POS_W * H

def _sc_body(ids_hbm, table_hbm, primes_hbm, y_hbm,
             idsbuf, idxbuf, rows, pbuf, sem):
    cid = lax.axis_index("c")
    sid = lax.axis_index("s")
    wid = sid * 2 + cid
    pos0 = wid * POS_W
    base = pos0 * H
    pltpu.sync_copy(ids_hbm.at[pl.ds(base, CH)], idsbuf.at[pl.ds(H, CH)])
    pltpu.sync_copy(primes_hbm, pbuf)
    row_start = (pos0 % L) == 0

    @pl.when(row_start)
    def _():
        idsbuf[pl.ds(0, H)] = jnp.zeros((H,), jnp.int32)

    @pl.when(jnp.logical_not(row_start))
    def _():
        pltpu.sync_copy(ids_hbm.at[pl.ds(base - H, H)], idsbuf.at[pl.ds(0, H)])

    lanes = lax.iota(jnp.int32, H)
    mult = lanes + 1
    primes = pbuf[...]
    offs = lanes * V

    def _hash_step(p, carry):
        cur = idsbuf[pl.ds(p * H + H, H)]
        prev = idsbuf[pl.ds(p * H, H)]
        bg = cur + prev * K
        r = ((bg * mult + mult) % primes) % V + offs
        w0 = (r >> 7) * 1024 + (r & 127)
        pos = lanes * E + p * (H * E)
        for c in range(E):
            plsc.store_scatter(idxbuf, [pos + c], w0 + c * 128)
        return carry

    lax.fori_loop(0, POS_W, _hash_step, 0)
    pltpu.async_copy(table_hbm.at[idxbuf], rows, sem).wait()
    pltpu.sync_copy(rows, y_hbm.at[pl.ds(base * E, CH * E)])


@functools.lru_cache(maxsize=1)
def _make_sc_hash_gather():
    mesh = plsc.VectorSubcoreMesh(core_axis_name="c", subcore_axis_name="s")
    return pl.kernel(
        _sc_body,
        out_type=jax.ShapeDtypeStruct((TH * H * E,), jnp.float32),
        mesh=mesh,
        scratch_types=[
            pltpu.VMEM((CH + H,), jnp.int32),
            pltpu.VMEM((CH * E,), jnp.int32),
            pltpu.VMEM((CH * E,), jnp.float32),
            pltpu.VMEM((H,), jnp.int32),
            pltpu.SemaphoreType.DMA,
        ],
        compiler_params=pltpu.CompilerParams(use_tc_tiling_on_sc=False,
                                             needs_layout_passes=False),
    )


TLC = 512


def _assemble_body(x_ref, y_ref, xs_ref, xb_ref, ys_ref, yb_ref, o_ref):
    y2 = y_ref[...]
    r = lax.broadcasted_iota(jnp.int32, (H * E, H * E), 0)
    c = lax.broadcasted_iota(jnp.int32, (H * E, H * E), 1)
    grp = jnp.where(r // E == c // E, 1.0 / E, 0.0)
    mu = lax.dot_general(y2, grp, (((1,), (0,)), ((), ())),
                         precision=lax.Precision.HIGHEST,
                         preferred_element_type=jnp.float32)
    ms = lax.dot_general(y2 * y2, grp, (((1,), (0,)), ((), ())),
                         precision=lax.Precision.HIGHEST,
                         preferred_element_type=jnp.float32)
    var = ms - mu * mu
    yn = (y2 - mu) * lax.rsqrt(var + EPS) * ys_ref[0, :] + yb_ref[0, :]
    lane = lax.broadcasted_iota(jnp.int32, (TLC, D), 1)
    for h in range(H):
        xh = x_ref[:, h * D:(h + 1) * D]
        mux = jnp.mean(xh, axis=1, keepdims=True)
        varx = jnp.mean(xh * xh, axis=1, keepdims=True) - mux * mux
        xn = ((xh - mux) * lax.rsqrt(varx + EPS)
              * xs_ref[0, h * D:(h + 1) * D] + xb_ref[0, h * D:(h + 1) * D])
        yh = pltpu.roll(yn, (D - E) - E * h, axis=1)
        o_ref[:, h * D:(h + 1) * D] = jnp.where(lane < D - E, xn, yh)


def _assemble_body_prev(x_ref, y_ref, xs_ref, xb_ref, ys_ref, yb_ref,
                        prev_ref, o_ref):
    del prev_ref
    _assemble_body(x_ref, y_ref, xs_ref, xb_ref, ys_ref, yb_ref, o_ref)


BPC = TH // TLC


def _make_assemble(s):
    in_specs = [
        pl.BlockSpec((TLC, H * D), lambda i: (s * BPC + i, 0)),
        pl.BlockSpec((TLC, H * E), lambda i: (i, 0)),
        pl.BlockSpec((1, H * D), lambda i: (0, 0)),
        pl.BlockSpec((1, H * D), lambda i: (0, 0)),
        pl.BlockSpec((1, H * E), lambda i: (0, 0)),
        pl.BlockSpec((1, H * E), lambda i: (0, 0)),
    ]
    body = _assemble_body
    aliases = {}
    if s > 0:
        in_specs.append(pl.BlockSpec((8, H * D), lambda i: (0, 0)))
        body = _assemble_body_prev
        aliases = {6: 0}
    return pl.pallas_call(
        body,
        grid=(BPC,),
        in_specs=in_specs,
        out_specs=pl.BlockSpec((TLC, H * D), lambda i: (s * BPC + i, 0)),
        out_shape=jax.ShapeDtypeStruct((T, H * D), jnp.float32),
        input_output_aliases=aliases,
    )


def kernel(x, means, ngram_table, ln_x_scale, ln_x_bias, ln_y_scale, ln_y_bias):
    x2 = x.reshape(T, H * D)
    meansT = jnp.swapaxes(means, 1, 2) * (-2.0)
    primes = jnp.asarray(_PRIMES)
    table_flat = ngram_table.reshape(V * H // 128, 128, E).transpose(0, 2, 1).reshape(-1)
    xs_ = ln_x_scale.reshape(1, H * D)
    xb_ = ln_x_bias.reshape(1, H * D)
    ys_ = ln_y_scale.reshape(1, H * E)
    yb_ = ln_y_bias.reshape(1, H * E)
    sc = _make_sc_hash_gather()
    ys = []
    for s in range(NS):
        ids = _make_pq_ids(s)(x2, meansT)
        ys.append(sc(ids.reshape(TH * H), table_flat, primes))
    out = None
    for s in range(NS):
        y2 = ys[s].reshape(TH, H * E)
        args = (x2, y2, xs_, xb_, ys_, yb_)
        if s > 0:
            args = args + (out,)
        out = _make_assemble(s)(*args)
    return out.reshape(B, L, H * D)
